# baseline (speedup 1.0000x reference)
import functools

import jax
import jax.numpy as jnp
from jax import lax
from jax.experimental import pallas as pl
from jax.experimental.pallas import tpu as pltpu
from jax.experimental.pallas import tpu_sc as plsc

_EMBED_DIM = 32
_GROUP = 1024


def _make_lookup(num_idx: int):
    info = plsc.get_sparse_core_info()
    n_cores, n_sub = info.num_cores, info.num_subcores
    n_workers = n_sub
    per_w = num_idx // n_workers
    n_groups = per_w // _GROUP
    n_pairs = n_groups // 2

    mesh = plsc.VectorSubcoreMesh(core_axis_name="c", subcore_axis_name="s")

    @functools.partial(
        pl.kernel,
        mesh=mesh,
        out_type=jax.ShapeDtypeStruct((num_idx, _EMBED_DIM), jnp.float32),
        scratch_types=[
            pltpu.VMEM((per_w,), jnp.int32),
            pltpu.VMEM((_GROUP, _EMBED_DIM), jnp.float32),
            pltpu.VMEM((_GROUP, _EMBED_DIM), jnp.float32),
            pltpu.SemaphoreType.DMA,
            pltpu.SemaphoreType.DMA,
            pltpu.SemaphoreType.DMA,
            pltpu.SemaphoreType.DMA,
        ],
        compiler_params=pltpu.CompilerParams(use_tc_tiling_on_sc=False),
    )
    def lookup(idx_hbm, table_hbm, out_hbm, idx_v, buf_a, buf_b,
               gsem_a, gsem_b, wsem_a, wsem_b):
        cid = lax.axis_index("c")

        @pl.when(cid == 0)
        def _():
            wid = lax.axis_index("s")
            base = wid * per_w

            def gather(g, buf, gsem):
                pltpu.async_copy(
                    table_hbm.at[idx_v.at[pl.ds(g * _GROUP, _GROUP)]], buf, gsem
                )

            def drain(buf, sem):
                pltpu.make_async_copy(
                    table_hbm.at[pl.ds(0, _GROUP)], buf, sem).wait()

            def writeback(g, buf, wsem):
                pltpu.async_copy(
                    buf, out_hbm.at[pl.ds(base + g * _GROUP, _GROUP)], wsem)

            def drain_wb(buf, sem):
                pltpu.make_async_copy(
                    buf, out_hbm.at[pl.ds(base, _GROUP)], sem).wait()

            pltpu.sync_copy(idx_hbm.at[pl.ds(base, per_w)], idx_v)
            gather(0, buf_a, gsem_a)
            gather(1, buf_b, gsem_b)

            def body(h, carry):
                g = 2 * h
                drain(buf_a, gsem_a)
                writeback(g, buf_a, wsem_a)
                drain(buf_b, gsem_b)
                writeback(g + 1, buf_b, wsem_b)
                drain_wb(buf_a, wsem_a)
                gather(g + 2, buf_a, gsem_a)
                drain_wb(buf_b, wsem_b)
                gather(g + 3, buf_b, gsem_b)
                return carry

            lax.fori_loop(0, n_pairs - 1, body, 0)

            g_last = n_groups - 2
            drain(buf_a, gsem_a)
            writeback(g_last, buf_a, wsem_a)
            drain(buf_b, gsem_b)
            writeback(g_last + 1, buf_b, wsem_b)
            drain_wb(buf_a, wsem_a)
            drain_wb(buf_b, wsem_b)

    return lookup


def kernel(token_ids, weight):
    s0, s1 = token_ids.shape
    num_idx = s0 * s1
    idx = token_ids.reshape(num_idx).astype(jnp.int32)
    out = _make_lookup(num_idx)(idx, weight)
    return out.reshape(s0, s1, _EMBED_DIM)


# speedup vs baseline: 1.0387x; 1.0387x over previous
import functools

import jax
import jax.numpy as jnp
from jax import lax
from jax.experimental import pallas as pl
from jax.experimental.pallas import tpu as pltpu
from jax.experimental.pallas import tpu_sc as plsc

_EMBED_DIM = 32
_GROUP = 1600


def _make_lookup(num_idx: int):
    info = plsc.get_sparse_core_info()
    n_cores, n_sub = info.num_cores, info.num_subcores
    n_workers = n_cores * n_sub
    per_w = num_idx // n_workers
    n_groups = per_w // _GROUP
    n_pairs = n_groups // 2

    mesh = plsc.VectorSubcoreMesh(core_axis_name="c", subcore_axis_name="s")

    @functools.partial(
        pl.kernel,
        mesh=mesh,
        out_type=jax.ShapeDtypeStruct((num_idx, _EMBED_DIM), jnp.float32),
        scratch_types=[
            pltpu.VMEM((per_w,), jnp.int32),
            pltpu.VMEM((_GROUP, _EMBED_DIM), jnp.float32),
            pltpu.VMEM((_GROUP, _EMBED_DIM), jnp.float32),
            pltpu.SemaphoreType.DMA,
            pltpu.SemaphoreType.DMA,
            pltpu.SemaphoreType.DMA,
            pltpu.SemaphoreType.DMA,
        ],
        compiler_params=pltpu.CompilerParams(use_tc_tiling_on_sc=False),
    )
    def lookup(idx_hbm, table_hbm, out_hbm, idx_v, buf_a, buf_b,
               gsem_a, gsem_b, wsem_a, wsem_b):
        wid = lax.axis_index("s") * n_cores + lax.axis_index("c")
        base = wid * per_w

        def gather(g, buf, gsem):
            pltpu.async_copy(
                table_hbm.at[idx_v.at[pl.ds(g * _GROUP, _GROUP)]], buf, gsem
            )

        def drain(buf, sem):
            pltpu.make_async_copy(table_hbm.at[pl.ds(0, _GROUP)], buf, sem).wait()

        def writeback(g, buf, wsem):
            pltpu.async_copy(buf, out_hbm.at[pl.ds(base + g * _GROUP, _GROUP)], wsem)

        def drain_wb(buf, sem):
            pltpu.make_async_copy(buf, out_hbm.at[pl.ds(base, _GROUP)], sem).wait()

        pltpu.sync_copy(idx_hbm.at[pl.ds(base, per_w)], idx_v)
        gather(0, buf_a, gsem_a)
        gather(1, buf_b, gsem_b)

        def body(h, carry):
            g = 2 * h
            drain(buf_a, gsem_a)
            writeback(g, buf_a, wsem_a)
            drain(buf_b, gsem_b)
            writeback(g + 1, buf_b, wsem_b)
            drain_wb(buf_a, wsem_a)
            gather(g + 2, buf_a, gsem_a)
            drain_wb(buf_b, wsem_b)
            gather(g + 3, buf_b, gsem_b)
            return carry

        lax.fori_loop(0, n_pairs - 1, body, 0)

        g_last = n_groups - 2
        drain(buf_a, gsem_a)
        writeback(g_last, buf_a, wsem_a)
        drain(buf_b, gsem_b)
        writeback(g_last + 1, buf_b, wsem_b)
        drain_wb(buf_a, wsem_a)
        drain_wb(buf_b, wsem_b)

    return lookup


def kernel(token_ids, weight):
    s0, s1 = token_ids.shape
    num_idx = s0 * s1
    idx = token_ids.reshape(num_idx).astype(jnp.int32)
    out = _make_lookup(num_idx)(idx, weight)
    return out.reshape(s0, s1, _EMBED_DIM)


# speedup vs baseline: 1.0388x; 1.0001x over previous
import functools

import jax
import jax.numpy as jnp
from jax import lax
from jax.experimental import pallas as pl
from jax.experimental.pallas import tpu as pltpu
from jax.experimental.pallas import tpu_sc as plsc

_EMBED_DIM = 32
_GROUP = 1600


def _make_lookup(num_idx: int):
    info = plsc.get_sparse_core_info()
    n_cores, n_sub = info.num_cores, info.num_subcores
    n_workers = n_cores * n_sub
    per_w = num_idx // n_workers
    n_groups = per_w // _GROUP
    n_pairs = n_groups // 2

    mesh = plsc.VectorSubcoreMesh(core_axis_name="c", subcore_axis_name="s")

    @functools.partial(
        pl.kernel,
        mesh=mesh,
        out_type=jax.ShapeDtypeStruct((num_idx, _EMBED_DIM), jnp.float32),
        scratch_types=[
            pltpu.VMEM((per_w,), jnp.int32),
            pltpu.VMEM((_GROUP, _EMBED_DIM), jnp.float32),
            pltpu.VMEM((_GROUP, _EMBED_DIM), jnp.float32),
            pltpu.SemaphoreType.DMA,
            pltpu.SemaphoreType.DMA,
            pltpu.SemaphoreType.DMA,
            pltpu.SemaphoreType.DMA,
            pltpu.SemaphoreType.DMA,
        ],
        compiler_params=pltpu.CompilerParams(use_tc_tiling_on_sc=False),
    )
    def lookup(idx_hbm, table_hbm, out_hbm, idx_v, buf_a, buf_b,
               gsem_a, gsem_b, wsem_a, wsem_b, isem):
        wid = lax.axis_index("s") * n_cores + lax.axis_index("c")
        base = wid * per_w

        def gather(g, buf, gsem):
            pltpu.async_copy(
                table_hbm.at[idx_v.at[pl.ds(g * _GROUP, _GROUP)]], buf, gsem
            )

        def drain(buf, sem):
            pltpu.make_async_copy(table_hbm.at[pl.ds(0, _GROUP)], buf, sem).wait()

        def writeback(g, buf, wsem):
            pltpu.async_copy(buf, out_hbm.at[pl.ds(base + g * _GROUP, _GROUP)], wsem)

        def drain_wb(buf, sem):
            pltpu.make_async_copy(buf, out_hbm.at[pl.ds(base, _GROUP)], sem).wait()

        prime = 2 * _GROUP
        pltpu.sync_copy(
            idx_hbm.at[pl.ds(base, prime)], idx_v.at[pl.ds(0, prime)]
        )
        gather(0, buf_a, gsem_a)
        gather(1, buf_b, gsem_b)
        pltpu.async_copy(
            idx_hbm.at[pl.ds(base + prime, per_w - prime)],
            idx_v.at[pl.ds(prime, per_w - prime)],
            isem,
        )
        pltpu.make_async_copy(
            idx_hbm.at[pl.ds(base + prime, per_w - prime)],
            idx_v.at[pl.ds(prime, per_w - prime)],
            isem,
        ).wait()

        def body(h, carry):
            g = 2 * h
            drain(buf_a, gsem_a)
            writeback(g, buf_a, wsem_a)
            drain(buf_b, gsem_b)
            writeback(g + 1, buf_b, wsem_b)
            drain_wb(buf_a, wsem_a)
            gather(g + 2, buf_a, gsem_a)
            drain_wb(buf_b, wsem_b)
            gather(g + 3, buf_b, gsem_b)
            return carry

        lax.fori_loop(0, n_pairs - 1, body, 0)

        g_last = n_groups - 2
        drain(buf_a, gsem_a)
        writeback(g_last, buf_a, wsem_a)
        drain(buf_b, gsem_b)
        writeback(g_last + 1, buf_b, wsem_b)
        drain_wb(buf_a, wsem_a)
        drain_wb(buf_b, wsem_b)

    return lookup


def kernel(token_ids, weight):
    s0, s1 = token_ids.shape
    num_idx = s0 * s1
    idx = token_ids.reshape(num_idx).astype(jnp.int32)
    out = _make_lookup(num_idx)(idx, weight)
    return out.reshape(s0, s1, _EMBED_DIM)
